# baseline (speedup 1.0000x reference)
import functools
import math

import jax
import jax.numpy as jnp
from jax import lax
from jax.experimental import pallas as pl
from jax.experimental.pallas import tpu as pltpu
from jax.experimental.pallas import tpu_sc as plsc

_CLIP_RATIO = 0.016
_MAX_RATIO = abs(math.log(_CLIP_RATIO))


def _tc_body(score_ref, logits_ref, logits_out_ref):
    for i in range(logits_ref.shape[0]):
        logits_out_ref[i] = logits_ref[i] * score_ref[i]


def _take16(x, idx):
    return lax.gather(
        x, idx[:, None],
        dimension_numbers=lax.GatherDimensionNumbers(
            offset_dims=(), collapsed_slice_dims=(0,), start_index_map=(0,)),
        slice_sizes=(1,),
        mode=lax.GatherScatterMode.PROMISE_IN_BOUNDS)


def _sc_boxes(regress_hbm, anchors_hbm, out_hbm, r_v, a_v, o_v):
    nc = 2
    w = lax.axis_index("s") * nc + lax.axis_index("c")
    chunk = 20000
    rbase = w * chunk
    abase = (w % 4) * chunk
    pltpu.sync_copy(regress_hbm.at[pl.ds(rbase, chunk)], r_v)
    pltpu.sync_copy(anchors_hbm.at[pl.ds(abase, chunk)], a_v)

    lane = lax.iota(jnp.int32, 16)
    comp = lane % 4
    quad = lane - comp
    half = comp % 2
    idx_ctr = quad + half
    idx_wh = quad + 2 + half
    sign_half = jnp.where(comp >= 2, 0.5, -0.5).astype(jnp.float32)

    def body(i, _):
        base = i * 16
        a16 = a_v[pl.ds(base, 16)]
        r16 = r_v[pl.ds(base, 16)]
        a_ctr = _take16(a16, idx_ctr)
        a_wh = _take16(a16, idx_wh)
        r_ctr = _take16(r16, idx_ctr)
        r_wh = _take16(r16, idx_wh)
        ctr = a_ctr + r_ctr * a_wh
        size = a_wh * jnp.exp(jnp.clip(r_wh, -_MAX_RATIO, _MAX_RATIO))
        box = jnp.clip(ctr + sign_half * size, 0.0, 1.0)
        o_v[pl.ds(base, 16)] = box
        return 0

    lax.fori_loop(0, chunk // 16, body, 0)
    pltpu.sync_copy(o_v, out_hbm.at[pl.ds(rbase, chunk)])


@functools.partial(jax.jit, static_argnames=("interpret",))
def kernel(score, logits, regress, anchors, interpret=False):
    B, N, C = logits.shape
    BB = 2

    logits_t = logits.transpose(0, 2, 1)
    score_t = score.transpose(0, 2, 1)

    logits_out_t = pl.pallas_call(
        _tc_body,
        grid=(B // BB,),
        in_specs=[
            pl.BlockSpec((BB, 1, N), lambda b: (b, 0, 0)),
            pl.BlockSpec((BB, C, N), lambda b: (b, 0, 0)),
        ],
        out_specs=pl.BlockSpec((BB, C, N), lambda b: (b, 0, 0)),
        out_shape=jax.ShapeDtypeStruct((B, C, N), jnp.float32),
        interpret=interpret,
    )(score_t, logits_t)

    mesh = plsc.VectorSubcoreMesh(core_axis_name="c", subcore_axis_name="s")
    boxes_flat = pl.kernel(
        _sc_boxes,
        mesh=mesh,
        out_type=jax.ShapeDtypeStruct((B * N * 4,), jnp.float32),
        scratch_types=[
            pltpu.VMEM((20000,), jnp.float32),
            pltpu.VMEM((20000,), jnp.float32),
            pltpu.VMEM((20000,), jnp.float32),
        ],
    )(regress.reshape(-1), anchors.reshape(-1))

    return logits_out_t.transpose(0, 2, 1), boxes_flat.reshape(B, N, 4)


# speedup vs baseline: 8.3379x; 8.3379x over previous
import functools
import math

import jax
import jax.numpy as jnp
from jax.experimental import pallas as pl

_CLIP_RATIO = 0.016
_MAX_RATIO = abs(math.log(_CLIP_RATIO))


def _body(score_ref, logits_ref, regress_ref, anchors_ref,
          logits_out_ref, boxes_out_ref):
    nb = logits_ref.shape[0]
    for i in range(nb):
        s = score_ref[i]
        logits_out_ref[i] = logits_ref[i] * s

    a = anchors_ref[...]
    for i in range(nb):
        r = regress_ref[i]
        cx = a[0:1] + r[0:1] * a[2:3]
        cy = a[1:2] + r[1:2] * a[3:4]
        w = a[2:3] * jnp.exp(jnp.clip(r[2:3], -_MAX_RATIO, _MAX_RATIO))
        h = a[3:4] * jnp.exp(jnp.clip(r[3:4], -_MAX_RATIO, _MAX_RATIO))
        x1 = cx - 0.5 * w
        y1 = cy - 0.5 * h
        x2 = cx + 0.5 * w
        y2 = cy + 0.5 * h
        boxes = jnp.concatenate([x1, y1, x2, y2], axis=0)
        boxes_out_ref[i] = jnp.clip(boxes, 0.0, 1.0)


@functools.partial(jax.jit, static_argnames=("interpret",))
def kernel(score, logits, regress, anchors, interpret=False):
    B, N, C = logits.shape
    BB = 2

    logits_t = logits.transpose(0, 2, 1)
    score_t = score.transpose(0, 2, 1)
    regress_t = regress.transpose(0, 2, 1)
    anchors_t = anchors.transpose(1, 0)

    logits_out_t, boxes_out_t = pl.pallas_call(
        _body,
        grid=(B // BB,),
        in_specs=[
            pl.BlockSpec((BB, 1, N), lambda b: (b, 0, 0)),
            pl.BlockSpec((BB, C, N), lambda b: (b, 0, 0)),
            pl.BlockSpec((BB, 4, N), lambda b: (b, 0, 0)),
            pl.BlockSpec((4, N), lambda b: (0, 0)),
        ],
        out_specs=[
            pl.BlockSpec((BB, C, N), lambda b: (b, 0, 0)),
            pl.BlockSpec((BB, 4, N), lambda b: (b, 0, 0)),
        ],
        out_shape=[
            jax.ShapeDtypeStruct((B, C, N), jnp.float32),
            jax.ShapeDtypeStruct((B, 4, N), jnp.float32),
        ],
        interpret=interpret,
    )(score_t, logits_t, regress_t, anchors_t)

    return logits_out_t.transpose(0, 2, 1), boxes_out_t.transpose(0, 2, 1)


# speedup vs baseline: 8.3496x; 1.0014x over previous
import math

import jax
import jax.numpy as jnp
from jax.experimental import pallas as pl

_CLIP_RATIO = 0.016
_MAX_RATIO = abs(math.log(_CLIP_RATIO))


def _body(score_ref, logits_ref, regress_ref, anchors_ref,
          logits_out_ref, boxes_out_ref):
    nb = logits_ref.shape[0]
    for i in range(nb):
        s = score_ref[i]
        logits_out_ref[i] = logits_ref[i] * s

    a = anchors_ref[...]
    for i in range(nb):
        r = regress_ref[i]
        cx = a[0:1] + r[0:1] * a[2:3]
        cy = a[1:2] + r[1:2] * a[3:4]
        w = a[2:3] * jnp.exp(jnp.clip(r[2:3], -_MAX_RATIO, _MAX_RATIO))
        h = a[3:4] * jnp.exp(jnp.clip(r[3:4], -_MAX_RATIO, _MAX_RATIO))
        x1 = cx - 0.5 * w
        y1 = cy - 0.5 * h
        x2 = cx + 0.5 * w
        y2 = cy + 0.5 * h
        boxes = jnp.concatenate([x1, y1, x2, y2], axis=0)
        boxes_out_ref[i] = jnp.clip(boxes, 0.0, 1.0)


@jax.jit
def kernel(score, logits, regress, anchors):
    B, N, C = logits.shape
    BB = 2

    logits_t = logits.transpose(0, 2, 1)
    score_t = score.transpose(0, 2, 1)
    regress_t = regress.transpose(0, 2, 1)
    anchors_t = anchors.transpose(1, 0)

    logits_out_t, boxes_out_t = pl.pallas_call(
        _body,
        grid=(B // BB,),
        in_specs=[
            pl.BlockSpec((BB, 1, N), lambda b: (b, 0, 0)),
            pl.BlockSpec((BB, C, N), lambda b: (b, 0, 0)),
            pl.BlockSpec((BB, 4, N), lambda b: (b, 0, 0)),
            pl.BlockSpec((4, N), lambda b: (0, 0)),
        ],
        out_specs=[
            pl.BlockSpec((BB, C, N), lambda b: (b, 0, 0)),
            pl.BlockSpec((BB, 4, N), lambda b: (b, 0, 0)),
        ],
        out_shape=[
            jax.ShapeDtypeStruct((B, C, N), jnp.float32),
            jax.ShapeDtypeStruct((B, 4, N), jnp.float32),
        ],
    )(score_t, logits_t, regress_t, anchors_t)

    return logits_out_t.transpose(0, 2, 1), boxes_out_t.transpose(0, 2, 1)
